# baseline (speedup 1.0000x reference)
import functools

import jax
import jax.numpy as jnp
from jax import lax
from jax.experimental import pallas as pl
from jax.experimental.pallas import tpu as pltpu
from jax.experimental.pallas import tpu_sc as plsc

BATCH = 131072
BINS = 256
EPS = 0.01
NC, NS, L = 2, 16, 16
NW = NC * NS
ROWS_PER_W = BATCH // NW
CHUNK = 32
NCHUNK = ROWS_PER_W // CHUNK
NV = BINS // L

_LN_P = (7.0376836292e-2, -1.1514610310e-1, 1.1676998740e-1, -1.2420140846e-1,
         1.4249322787e-1, -1.6668057665e-1, 2.0000714765e-1, -2.4999993993e-1,
         3.3333331174e-1)


def _ln16(u):
    i = lax.bitcast_convert_type(u, jnp.int32)
    e = ((i >> 23) & 0xFF) - 126
    m = lax.bitcast_convert_type((i & 0x007FFFFF) | 0x3F000000, jnp.float32)
    big = m > 0.70710678
    e = jnp.where(big, e, e - 1).astype(jnp.float32)
    x = jnp.where(big, m - 1.0, m + m - 1.0)
    z = x * x
    y = jnp.full((L,), _LN_P[0], jnp.float32)
    for p in _LN_P[1:]:
        y = y * x + p
    y = y * x * z
    y = y + e * (-2.12194440e-4)
    y = y - 0.5 * z
    return x + y + e * 0.693359375


def _sc_body(logits_hbm, u_hbm, out_hbm, lg_v, u_v, out_v):
    wid = lax.axis_index("s") * NC + lax.axis_index("c")
    iota = lax.iota(jnp.int32, L)
    neg_big = jnp.full((L,), -3.0e38, jnp.float32)

    def row_body(r, _):
        m16 = lg_v[r, pl.ds(0, L)]
        for j in range(1, NV):
            m16 = jnp.maximum(m16, lg_v[r, pl.ds(j * L, L)])
        m = jnp.max(m16)
        s16 = jnp.zeros((L,), jnp.float32)
        for j in range(NV):
            e = jnp.exp(lg_v[r, pl.ds(j * L, L)] - m)
            lg_v[r, pl.ds(j * L, L)] = e
            s16 = s16 + e
        c = jnp.sum(s16) * (EPS / BINS)
        cmax = neg_big
        cidx = jnp.zeros((L,), jnp.int32)
        for j in range(NV):
            t = (1.0 - EPS) * lg_v[r, pl.ds(j * L, L)] + c
            sc = t / (-_ln16(u_v[r, pl.ds(j * L, L)]))
            upd = sc > cmax
            cmax = jnp.where(upd, sc, cmax)
            cidx = jnp.where(upd, iota + (j * L), cidx)
        best = jnp.max(cmax)
        bidx = jnp.min(jnp.where(cmax == best, cidx, BINS))
        for j in range(NV):
            out_v[r, pl.ds(j * L, L)] = (iota + (j * L) == bidx).astype(jnp.float32)
        return 0

    def chunk_body(k, _):
        base = wid * ROWS_PER_W + k * CHUNK
        pltpu.sync_copy(logits_hbm.at[pl.ds(base, CHUNK)], lg_v)
        pltpu.sync_copy(u_hbm.at[pl.ds(base, CHUNK)], u_v)
        lax.fori_loop(0, CHUNK, row_body, 0)
        pltpu.sync_copy(out_v, out_hbm.at[pl.ds(base, CHUNK)])
        return 0

    lax.fori_loop(0, NCHUNK, chunk_body, 0)


INTERPRET = False


def kernel(logits, u):
    mesh = plsc.VectorSubcoreMesh(core_axis_name="c", subcore_axis_name="s")
    f = pl.kernel(
        _sc_body,
        mesh=mesh,
        interpret=INTERPRET,
        compiler_params=pltpu.CompilerParams(needs_layout_passes=False),
        out_type=jax.ShapeDtypeStruct((BATCH, BINS), jnp.float32),
        scratch_types=[
            pltpu.VMEM((CHUNK, BINS), jnp.float32),
            pltpu.VMEM((CHUNK, BINS), jnp.float32),
            pltpu.VMEM((CHUNK, BINS), jnp.float32),
        ],
    )
    return f(logits, u)


# speedup vs baseline: 8.0354x; 8.0354x over previous
import functools

import jax
import jax.numpy as jnp
from jax.experimental import pallas as pl
from jax.experimental.pallas import tpu as pltpu

BATCH = 131072
BINS = 256
EPS = 0.01
BLOCK_ROWS = 4096


def _tc_body(logits_ref, u_ref, out_ref):
    x = logits_ref[...]
    m = jnp.max(x, axis=-1, keepdims=True)
    e = jnp.exp(x - m)
    s = jnp.sum(e, axis=-1, keepdims=True)
    t = (1.0 - EPS) * e + (EPS / BINS) * s
    score = t / (-jnp.log(u_ref[...]))
    best = jnp.max(score, axis=-1, keepdims=True)
    iota = jax.lax.broadcasted_iota(jnp.int32, score.shape, 1)
    idx = jnp.min(jnp.where(score == best, iota, BINS), axis=-1, keepdims=True)
    out_ref[...] = (iota == idx).astype(jnp.float32)


def kernel(logits, u):
    grid = (BATCH // BLOCK_ROWS,)
    return pl.pallas_call(
        _tc_body,
        grid=grid,
        in_specs=[
            pl.BlockSpec((BLOCK_ROWS, BINS), lambda i: (i, 0)),
            pl.BlockSpec((BLOCK_ROWS, BINS), lambda i: (i, 0)),
        ],
        out_specs=pl.BlockSpec((BLOCK_ROWS, BINS), lambda i: (i, 0)),
        out_shape=jax.ShapeDtypeStruct((BATCH, BINS), jnp.float32),
    )(logits, u)
